# baseline (speedup 1.0000x reference)
import functools

import jax
import jax.numpy as jnp
from jax import lax
from jax.experimental import pallas as pl
from jax.experimental.pallas import tpu as pltpu
from jax.experimental.pallas import tpu_sc as plsc

B = 16384
D = 32
NC = 2
NS = 16
NW = NC * NS
BPW = B // NW

_sc_mesh = plsc.VectorSubcoreMesh(core_axis_name="c", subcore_axis_name="s")


@functools.partial(
    pl.kernel,
    mesh=_sc_mesh,
    compiler_params=pltpu.CompilerParams(use_tc_tiling_on_sc=False),
    out_type=[jax.ShapeDtypeStruct((D, B), jnp.float32) for _ in range(4)],
    scratch_types=[
        pltpu.VMEM((BPW,), jnp.int32),
        pltpu.VMEM((BPW,), jnp.int32),
        pltpu.VMEM((D, BPW), jnp.float32),
        pltpu.VMEM((D, BPW), jnp.float32),
        pltpu.VMEM((D, BPW), jnp.float32),
        pltpu.VMEM((D, BPW), jnp.float32),
        pltpu.SemaphoreType.DMA,
        pltpu.SemaphoreType.DMA,
        pltpu.SemaphoreType.DMA,
        pltpu.SemaphoreType.DMA,
    ],
)
def _sc_gather(user_hbm, item_hbm, ugt, igt, umt, imt,
               ug_o, ig_o, um_o, im_o,
               uidx, iidx, ug_v, ig_v, um_v, im_v, s0, s1, s2, s3):
    wid = lax.axis_index("s") * NC + lax.axis_index("c")
    base = wid * BPW
    pltpu.sync_copy(user_hbm.at[pl.ds(base, BPW)], uidx)
    pltpu.sync_copy(item_hbm.at[pl.ds(base, BPW)], iidx)
    for c in range(D):
        pltpu.async_copy(ugt.at[c].at[uidx], ug_v.at[c], s0)
        pltpu.async_copy(igt.at[c].at[iidx], ig_v.at[c], s1)
        pltpu.async_copy(umt.at[c].at[uidx], um_v.at[c], s2)
        pltpu.async_copy(imt.at[c].at[iidx], im_v.at[c], s3)
    for c in range(D):
        pltpu.make_async_copy(ugt.at[c].at[uidx], ug_v.at[c], s0).wait()
        pltpu.make_async_copy(igt.at[c].at[iidx], ig_v.at[c], s1).wait()
        pltpu.make_async_copy(umt.at[c].at[uidx], um_v.at[c], s2).wait()
        pltpu.make_async_copy(imt.at[c].at[iidx], im_v.at[c], s3).wait()
    pltpu.sync_copy(ug_v, ug_o.at[:, pl.ds(base, BPW)])
    pltpu.sync_copy(ig_v, ig_o.at[:, pl.ds(base, BPW)])
    pltpu.sync_copy(um_v, um_o.at[:, pl.ds(base, BPW)])
    pltpu.sync_copy(im_v, im_o.at[:, pl.ds(base, BPW)])


_BLK = 2048


def _mlp_body(ug, ig, um, im, w1a, w1b, b1, w2, b2, wpg, wph, bp, out):
    gmf = ug[...] * ig[...]
    h = (jnp.dot(w1a[...], um[...], preferred_element_type=jnp.float32)
         + jnp.dot(w1b[...], im[...], preferred_element_type=jnp.float32)
         + b1[...])
    h = jnp.maximum(h, 0.0)
    h = jnp.dot(w2[...], h, preferred_element_type=jnp.float32) + b2[...]
    h = jnp.maximum(h, 0.0)
    y = (jnp.dot(wpg[...], gmf[...], preferred_element_type=jnp.float32)
         + jnp.dot(wph[...], h[...], preferred_element_type=jnp.float32)
         + bp[...])
    out[...] = y


_col_spec = pl.BlockSpec((D, _BLK), lambda i: (0, i))


def _full(shape):
    return pl.BlockSpec(shape, lambda i: tuple(0 for _ in shape))


_mlp_call = pl.pallas_call(
    _mlp_body,
    grid=(B // _BLK,),
    in_specs=[
        _col_spec, _col_spec, _col_spec, _col_spec,
        _full((D, D)), _full((D, D)), _full((D, 1)),
        _full((16, D)), _full((16, 1)),
        _full((1, D)), _full((1, 16)), _full((1, 1)),
    ],
    out_specs=pl.BlockSpec((1, _BLK), lambda i: (0, i)),
    out_shape=jax.ShapeDtypeStruct((1, B), jnp.float32),
)


def kernel(user, item, user_gmf, item_gmf, user_mlp, item_mlp,
           W1, b1, W2, b2, Wp, bp):
    user = user.astype(jnp.int32)
    item = item.astype(jnp.int32)
    ug, ig, um, im = _sc_gather(user, item,
                                user_gmf.T, item_gmf.T,
                                user_mlp.T, item_mlp.T)
    y = _mlp_call(ug, ig, um, im,
                  W1[:D].T, W1[D:].T, b1.reshape(D, 1),
                  W2.T, b2.reshape(16, 1),
                  Wp[:D].T, Wp[D:].T, bp.reshape(1, 1))
    return y.reshape(B)


# speedup vs baseline: 18.8683x; 18.8683x over previous
import functools

import jax
import jax.numpy as jnp
from jax import lax
from jax.experimental import pallas as pl
from jax.experimental.pallas import tpu as pltpu
from jax.experimental.pallas import tpu_sc as plsc

B = 16384
D = 32
NC = 2
NS = 16
NW = NC * NS
BPW = B // NW
L = 16

_sc_mesh = plsc.VectorSubcoreMesh(core_axis_name="c", subcore_axis_name="s")


@functools.partial(
    pl.kernel,
    mesh=_sc_mesh,
    compiler_params=pltpu.CompilerParams(needs_layout_passes=False),
    out_type=[jax.ShapeDtypeStruct((D, B), jnp.float32) for _ in range(4)],
    scratch_types=[
        pltpu.VMEM((BPW,), jnp.int32),
        pltpu.VMEM((BPW,), jnp.int32),
        pltpu.VMEM((2, 4, D, 128), jnp.float32),
        pltpu.VMEM((D, BPW), jnp.float32),
        pltpu.VMEM((D, BPW), jnp.float32),
        pltpu.VMEM((D, BPW), jnp.float32),
        pltpu.VMEM((D, BPW), jnp.float32),
        pltpu.SemaphoreType.DMA,
        pltpu.SemaphoreType.DMA,
    ],
)
def _sc_gather(user_hbm, item_hbm, ugt, igt, umt, imt,
               ug_o, ig_o, um_o, im_o,
               uidx, iidx, slabs, st0, st1, st2, st3, sem0, sem1):
    wid = lax.axis_index("s") * NC + lax.axis_index("c")
    base = wid * BPW
    pltpu.sync_copy(user_hbm.at[pl.ds(base, BPW)], uidx)
    pltpu.sync_copy(item_hbm.at[pl.ds(base, BPW)], iidx)

    tabs = (ugt, umt, igt, imt)
    stages = (st0, st2, st1, st3)
    sems = (sem0, sem1)
    row_lo = lax.broadcasted_iota(jnp.int32, (L,), 0)
    row_hi = row_lo + L

    def slab_off(r):
        return pl.multiple_of((r // 128) * 128, 128)

    def fire(ru, ri, slot):
        sem = sems[slot]
        ou, oi = slab_off(ru), slab_off(ri)
        offs = (ou, ou, oi, oi)
        for t in range(4):
            pltpu.async_copy(tabs[t].at[:, pl.ds(offs[t], 128)],
                             slabs.at[slot, t], sem)

    def wait(slot):
        sem = sems[slot]
        for t in range(4):
            pltpu.make_async_copy(tabs[t].at[:, pl.ds(0, 128)],
                                  slabs.at[slot, t], sem).wait()

    def extract(j, ru, ri, slot):
        js = jnp.full((L,), j, jnp.int32)
        lanes = (ru % 128, ru % 128, ri % 128, ri % 128)
        for t in range(4):
            lv = jnp.full((L,), lanes[t], jnp.int32)
            v0 = plsc.load_gather(slabs.at[slot, t], [row_lo, lv])
            v1 = plsc.load_gather(slabs.at[slot, t], [row_hi, lv])
            plsc.store_scatter(stages[t], [row_lo, js], v0)
            plsc.store_scatter(stages[t], [row_hi, js], v1)

    uv0 = uidx[pl.ds(0, L)]
    iv0 = iidx[pl.ds(0, L)]
    fire(uv0[0], iv0[0], 0)

    def group_body(g, carry):
        g_nxt = jnp.minimum(g + 1, BPW // L - 1)
        uv = uidx[pl.ds(g * L, L)]
        iv = iidx[pl.ds(g * L, L)]
        uvn = uidx[pl.ds(g_nxt * L, L)]
        ivn = iidx[pl.ds(g_nxt * L, L)]
        for lane in range(L):
            j = g * L + lane
            slot = lane & 1
            if lane < L - 1:
                run, rin = uv[lane + 1], iv[lane + 1]
            else:
                run, rin = uvn[0], ivn[0]
            fire(run, rin, 1 - slot)
            wait(slot)
            extract(j, uv[lane], iv[lane], slot)
        return carry

    lax.fori_loop(0, BPW // L, group_body, 0)
    wait(0)

    pltpu.sync_copy(st0, ug_o.at[:, pl.ds(base, BPW)])
    pltpu.sync_copy(st1, ig_o.at[:, pl.ds(base, BPW)])
    pltpu.sync_copy(st2, um_o.at[:, pl.ds(base, BPW)])
    pltpu.sync_copy(st3, im_o.at[:, pl.ds(base, BPW)])


_BLK = 2048


def _mlp_body(ug, ig, um, im, w1a, w1b, b1, w2, b2, wpg, wph, bp, out):
    gmf = ug[...] * ig[...]
    h = (jnp.dot(w1a[...], um[...], preferred_element_type=jnp.float32)
         + jnp.dot(w1b[...], im[...], preferred_element_type=jnp.float32)
         + b1[...])
    h = jnp.maximum(h, 0.0)
    h = jnp.dot(w2[...], h, preferred_element_type=jnp.float32) + b2[...]
    h = jnp.maximum(h, 0.0)
    y = (jnp.dot(wpg[...], gmf[...], preferred_element_type=jnp.float32)
         + jnp.dot(wph[...], h[...], preferred_element_type=jnp.float32)
         + bp[...])
    out[...] = y


_col_spec = pl.BlockSpec((D, _BLK), lambda i: (0, i))


def _full(shape):
    return pl.BlockSpec(shape, lambda i: tuple(0 for _ in shape))


_mlp_call = pl.pallas_call(
    _mlp_body,
    grid=(B // _BLK,),
    in_specs=[
        _col_spec, _col_spec, _col_spec, _col_spec,
        _full((D, D)), _full((D, D)), _full((D, 1)),
        _full((16, D)), _full((16, 1)),
        _full((1, D)), _full((1, 16)), _full((1, 1)),
    ],
    out_specs=pl.BlockSpec((1, _BLK), lambda i: (0, i)),
    out_shape=jax.ShapeDtypeStruct((1, B), jnp.float32),
)


def kernel(user, item, user_gmf, item_gmf, user_mlp, item_mlp,
           W1, b1, W2, b2, Wp, bp):
    user = user.astype(jnp.int32)
    item = item.astype(jnp.int32)
    ug, ig, um, im = _sc_gather(user, item,
                                user_gmf.T, item_gmf.T,
                                user_mlp.T, item_mlp.T)
    y = _mlp_call(ug, ig, um, im,
                  W1[:D].T, W1[D:].T, b1.reshape(D, 1),
                  W2.T, b2.reshape(16, 1),
                  Wp[:D].T, Wp[D:].T, bp.reshape(1, 1))
    return y.reshape(B)
